# baseline (speedup 1.0000x reference)
import functools

import jax
import jax.numpy as jnp
from jax import lax
from jax.experimental import pallas as pl
from jax.experimental.pallas import tpu as pltpu
from jax.experimental.pallas import tpu_sc as plsc

H = 768
S = 77
SP = 80
B = 4096
NC, NS = 2, 16
SEQ_PER_SUB = B // NS
LANES = 16
NR = SP // NC
CH = 32
NBUF = 3
LOOP_SLOTS = SEQ_PER_SUB - 1

_mesh = plsc.VectorSubcoreMesh(core_axis_name="c", subcore_axis_name="s")


@functools.partial(
    pl.kernel,
    out_type=jax.ShapeDtypeStruct((B, SP, H), jnp.float32),
    mesh=_mesh,
    compiler_params=pltpu.CompilerParams(use_tc_tiling_on_sc=False),
    scratch_types=[
        pltpu.VMEM((2, CH, 1, SP), jnp.int32),
        pltpu.VMEM((NR, H), jnp.float32),
        pltpu.VMEM((NR, H), jnp.float32),
        pltpu.VMEM((NR, H), jnp.float32),
        pltpu.VMEM((NR, H), jnp.float32),
        pltpu.SemaphoreType.DMA,
        pltpu.SemaphoreType.DMA,
        pltpu.SemaphoreType.DMA,
        pltpu.SemaphoreType.DMA,
        pltpu.SemaphoreType.DMA,
        pltpu.SemaphoreType.DMA,
    ],
)
def _embed(ids_hbm, tab_hbm, pos_hbm, out_hbm,
           idx_v, pos_v, buf0, buf1, buf2, g0, g1, g2, so0, so1, so2):
    c = lax.axis_index("c")
    sid = lax.axis_index("s")
    seq0 = sid * SEQ_PER_SUB
    r0 = c * NR

    bufs = (buf0, buf1, buf2)
    gsem = (g0, g1, g2)
    ssem = (so0, so1, so2)

    pltpu.sync_copy(pos_hbm.at[pl.ds(r0, NR)], pos_v)

    def refresh(j):
        pltpu.sync_copy(ids_hbm.at[pl.ds(seq0 + j, CH)],
                        idx_v.at[(j // CH) % 2])

    def gstart(j, b):
        pltpu.async_copy(
            tab_hbm.at[idx_v.at[(j // CH) % 2, j % CH, 0, pl.ds(r0, NR)]],
            bufs[b], gsem[b])

    def gwait(b):
        pltpu.make_async_copy(
            tab_hbm.at[pl.ds(0, NR)], bufs[b], gsem[b]).wait()

    def sstart(j, b):
        pltpu.async_copy(
            bufs[b], out_hbm.at[seq0 + j, pl.ds(r0, NR)], ssem[b])

    def swait(j, b):
        pltpu.make_async_copy(
            bufs[b], out_hbm.at[seq0 + j, pl.ds(r0, NR)], ssem[b]).wait()

    def add_pos(b):
        def add_row(r, c2):
            for g in range(H // LANES):
                sl = pl.ds(g * LANES, LANES)
                plsc.addupdate(bufs[b].at[r, sl], pos_v[r, sl])
            return c2
        lax.fori_loop(0, NR, add_row, 0)

    refresh(0)
    gstart(0, 0)
    gstart(1, 1)

    def outer(i2, carry):
        for b in range(NBUF):
            i = i2 * NBUF + b
            nb = (b + 2) % NBUF
            gwait(b)

            @pl.when(i >= 1)
            def _():
                swait(i - 1, nb)

            @pl.when(jnp.logical_and((i + 2) % CH == 0,
                                     i + 2 < LOOP_SLOTS))
            def _():
                refresh(i + 2)

            @pl.when(i + 2 <= LOOP_SLOTS - 1)
            def _():
                gstart(i + 2, nb)

            add_pos(b)
            sstart(i, b)
        return carry

    lax.fori_loop(0, LOOP_SLOTS // NBUF, outer, 0)

    gstart(SEQ_PER_SUB - 1, 0)
    gwait(0)
    add_pos(0)
    sstart(SEQ_PER_SUB - 1, 0)
    swait(SEQ_PER_SUB - 2, 2)
    swait(SEQ_PER_SUB - 1, 0)


def kernel(input_ids, token_embedding, position_embedding):
    ids_pad = jnp.pad(input_ids, ((0, 0), (0, SP - S))).reshape(B, 1, SP)
    pos_pad = jnp.pad(position_embedding, ((0, SP - S), (0, 0)))
    out = _embed(ids_pad, token_embedding, pos_pad)
    return out[:, :S, :]


# speedup vs baseline: 1.5095x; 1.5095x over previous
import functools

import jax
import jax.numpy as jnp
from jax import lax
from jax.experimental import pallas as pl
from jax.experimental.pallas import tpu as pltpu
from jax.experimental.pallas import tpu_sc as plsc

H = 768
S = 77
SP = 80
B = 4096
NC, NS = 2, 16
SEQ_PER_SUB = B // NS
LANES = 16
NR = SP // NC
IDX_PER_W = SEQ_PER_SUB * NR

_mesh = plsc.VectorSubcoreMesh(core_axis_name="c", subcore_axis_name="s")


@functools.partial(
    pl.kernel,
    out_type=jax.ShapeDtypeStruct((B, SP, H), jnp.float32),
    mesh=_mesh,
    scratch_types=[
        pltpu.VMEM((IDX_PER_W,), jnp.int32),
        pltpu.VMEM((NR, H), jnp.float32),
        pltpu.VMEM((NR, H), jnp.float32),
        pltpu.VMEM((NR, H), jnp.float32),
        pltpu.SemaphoreType.DMA,
        pltpu.SemaphoreType.DMA,
        pltpu.SemaphoreType.DMA,
        pltpu.SemaphoreType.DMA,
    ],
)
def _embed(idsw_hbm, tab_hbm, pos_hbm, out_hbm,
           idx_v, pos_v, buf0, buf1, g0, g1, so0, so1):
    c = lax.axis_index("c")
    sid = lax.axis_index("s")
    seq0 = sid * SEQ_PER_SUB
    woff = pl.multiple_of((sid * NC + c) * IDX_PER_W, 8)
    pltpu.sync_copy(idsw_hbm.at[pl.ds(woff, IDX_PER_W)], idx_v)

    bufs = (buf0, buf1)
    gsem = (g0, g1)
    ssem = (so0, so1)

    for ci in range(NC):
        r0 = ci * NR

        @pl.when(c == ci)
        def _():
            pltpu.sync_copy(pos_hbm.at[pl.ds(r0, NR)], pos_v)

            def gstart(i, b):
                off = pl.multiple_of(i * NR, 8)
                pltpu.async_copy(
                    tab_hbm.at[idx_v.at[pl.ds(off, NR)]], bufs[b], gsem[b])

            def gwait(b):
                pltpu.make_async_copy(
                    tab_hbm.at[pl.ds(0, NR)], bufs[b], gsem[b]).wait()

            def sstart(i, b):
                pltpu.async_copy(
                    bufs[b], out_hbm.at[seq0 + i, pl.ds(r0, NR)], ssem[b])

            def swait(i, b):
                pltpu.make_async_copy(
                    bufs[b], out_hbm.at[seq0 + i, pl.ds(r0, NR)],
                    ssem[b]).wait()

            gstart(0, 0)

            def outer(i2, carry):
                for b in range(2):
                    ob = 1 - b
                    i = i2 * 2 + b
                    gwait(b)
                    if b == 0:
                        @pl.when(i2 >= 1)
                        def _():
                            swait(i - 1, ob)
                        gstart(i + 1, ob)
                    else:
                        swait(i - 1, ob)
                        @pl.when(i2 <= (SEQ_PER_SUB // 2) - 2)
                        def _():
                            gstart(i + 1, ob)

                    def add_row(r, c2):
                        for g in range(H // LANES):
                            sl = pl.ds(g * LANES, LANES)
                            bufs[b][r, sl] = bufs[b][r, sl] + pos_v[r, sl]
                        return c2

                    lax.fori_loop(0, NR, add_row, 0)
                    sstart(i, b)
                return carry

            lax.fori_loop(0, SEQ_PER_SUB // 2, outer, 0)
            swait(SEQ_PER_SUB - 1, 1)


def kernel(input_ids, token_embedding, position_embedding):
    ids_pad = jnp.pad(input_ids, ((0, 0), (0, SP - S)))
    ids_w = (ids_pad.reshape(NS, SEQ_PER_SUB, NC, NR)
             .transpose(0, 2, 1, 3).reshape(-1))
    pos_pad = jnp.pad(position_embedding, ((0, SP - S), (0, 0)))
    out = _embed(ids_w, token_embedding, pos_pad)
    return out[:, :S, :]
